# Initial kernel scaffold; baseline (speedup 1.0000x reference)
import functools

import jax
import jax.numpy as jnp
from jax import lax
from jax.experimental import pallas as pl
from jax.experimental.pallas import tpu as pltpu
from jax.experimental.pallas import tpu_sc as plsc

N = 10000
IN_CH = 128
HID = 64
OUT_PAD = 16
E = 320000

NC = 2
NS = 16
NW = NC * NS
EPT = E // NW
CH = 100
NCHUNK = EPT // CH
RPT = N // NS

BLK = 1000


def _mesh():
    return plsc.VectorSubcoreMesh(
        core_axis_name="c", subcore_axis_name="s",
        num_cores=NC, num_subcores=NS)


@functools.partial(
    pl.kernel,
    out_type=(jax.ShapeDtypeStruct((NC, N, 16), jnp.float32),
              jax.ShapeDtypeStruct((NC, N, 16), jnp.float32)),
    mesh=_mesh(),
    scratch_types=[
        pltpu.VMEM((NCHUNK, CH), jnp.int32),
        pltpu.VMEM((CH, 16), jnp.float32),
        pltpu.VMEM((RPT, 16), jnp.float32),
        pltpu.VMEM_SHARED((N, 16), jnp.float32),
        pltpu.VMEM_SHARED((N, 16), jnp.float32),
    ],
)
def _sc_degree(dst_sp, dst_si, ones_in, zeros_in, degp_sp, degp_si,
               dst_v, ones_v, stage_v, acc_sp, acc_si):
    c = lax.axis_index("c")
    s = lax.axis_index("s")
    wid = c * NS + s
    row0 = s * RPT
    pltpu.sync_copy(zeros_in, stage_v)
    pltpu.sync_copy(stage_v, acc_sp.at[pl.ds(row0, RPT)])
    pltpu.sync_copy(stage_v, acc_si.at[pl.ds(row0, RPT)])
    pltpu.sync_copy(ones_in, ones_v)
    plsc.subcore_barrier()
    for dst_hbm, acc in ((dst_sp, acc_sp), (dst_si, acc_si)):
        pltpu.sync_copy(dst_hbm.at[wid], dst_v)

        @pl.loop(0, NCHUNK)
        def _(i, acc=acc):
            pltpu.sync_copy(ones_v, acc.at[dst_v.at[i]], add=True)

    plsc.subcore_barrier()
    for acc, out in ((acc_sp, degp_sp), (acc_si, degp_si)):
        pltpu.sync_copy(acc.at[pl.ds(row0, RPT)], stage_v)
        pltpu.sync_copy(stage_v, out.at[c, pl.ds(row0, RPT)])


def _make_sc_agg(width):
    @functools.partial(
        pl.kernel,
        out_type=(jax.ShapeDtypeStruct((NC, N, width), jnp.float32),
                  jax.ShapeDtypeStruct((NC, N, width), jnp.float32)),
        mesh=_mesh(),
        scratch_types=[
            pltpu.VMEM((NCHUNK, CH), jnp.int32),
            pltpu.VMEM((NCHUNK, CH), jnp.int32),
            pltpu.VMEM((CH, width), jnp.float32),
            pltpu.VMEM((CH, width), jnp.float32),
            pltpu.VMEM((RPT, width), jnp.float32),
            pltpu.VMEM_SHARED((N, width), jnp.float32),
            pltpu.VMEM_SHARED((N, width), jnp.float32),
            pltpu.SemaphoreType.DMA,
            pltpu.SemaphoreType.DMA,
        ],
    )
    def _sc_agg(src_sp, dst_sp, src_si, dst_si, y_sp, y_si, zeros_in,
                aggp_sp, aggp_si,
                src_v, dst_v, rows0, rows1, stage_v, acc_sp, acc_si,
                sem0, sem1):
        c = lax.axis_index("c")
        s = lax.axis_index("s")
        wid = c * NS + s
        row0 = s * RPT
        pltpu.sync_copy(zeros_in, stage_v)
        pltpu.sync_copy(stage_v, acc_sp.at[pl.ds(row0, RPT)])
        pltpu.sync_copy(stage_v, acc_si.at[pl.ds(row0, RPT)])
        plsc.subcore_barrier()
        for src_hbm, dst_hbm, y_hbm, acc in (
                (src_sp, dst_sp, y_sp, acc_sp),
                (src_si, dst_si, y_si, acc_si)):
            pltpu.sync_copy(src_hbm.at[wid], src_v)
            pltpu.sync_copy(dst_hbm.at[wid], dst_v)

            @pl.loop(0, NCHUNK, step=2)
            def _(i, y_hbm=y_hbm, acc=acc):
                cp0 = pltpu.async_copy(y_hbm.at[src_v.at[i]], rows0, sem0)
                cp1 = pltpu.async_copy(y_hbm.at[src_v.at[i + 1]], rows1, sem1)
                cp0.wait()
                pltpu.sync_copy(rows0, acc.at[dst_v.at[i]], add=True)
                cp1.wait()
                pltpu.sync_copy(rows1, acc.at[dst_v.at[i + 1]], add=True)

        plsc.subcore_barrier()
        for acc, out in ((acc_sp, aggp_sp), (acc_si, aggp_si)):
            pltpu.sync_copy(acc.at[pl.ds(row0, RPT)], stage_v)
            pltpu.sync_copy(stage_v, out.at[c, pl.ds(row0, RPT)])

    return _sc_agg


_sc_agg64 = _make_sc_agg(HID)
_sc_agg16 = _make_sc_agg(OUT_PAD)


def _dinvs(dp_ref):
    deg = dp_ref[0, :, 0:1] + dp_ref[1, :, 0:1] + 1.0
    return lax.rsqrt(deg)


def _tc_pre_body(x_ref, w_sp_ref, w_si_ref, dp_sp_ref, dp_si_ref,
                 y_sp_ref, y_si_ref):
    xb = x_ref[...]
    y_sp_ref[...] = _dinvs(dp_sp_ref) * jnp.dot(
        xb, w_sp_ref[...], preferred_element_type=jnp.float32)
    y_si_ref[...] = _dinvs(dp_si_ref) * jnp.dot(
        xb, w_si_ref[...], preferred_element_type=jnp.float32)


def _tc_mid_body(ap_sp_ref, ap_si_ref, y_sp_ref, y_si_ref,
                 dp_sp_ref, dp_si_ref, b1_sp_ref, b1_si_ref,
                 w2_sp_ref, w2_si_ref, z_sp_ref, z_si_ref):
    di_sp = _dinvs(dp_sp_ref)
    di_si = _dinvs(dp_si_ref)
    h_sp = di_sp * (ap_sp_ref[0] + ap_sp_ref[1] + y_sp_ref[...]) + b1_sp_ref[...]
    h_si = di_si * (ap_si_ref[0] + ap_si_ref[1] + y_si_ref[...]) + b1_si_ref[...]
    h = jnp.maximum((h_sp + h_si) * 0.5, 0.0)
    z_sp_ref[...] = di_sp * jnp.dot(
        h, w2_sp_ref[...], preferred_element_type=jnp.float32)
    z_si_ref[...] = di_si * jnp.dot(
        h, w2_si_ref[...], preferred_element_type=jnp.float32)


def _tc_final_body(ap_sp_ref, ap_si_ref, z_sp_ref, z_si_ref,
                   dp_sp_ref, dp_si_ref, b2_sp_ref, b2_si_ref, out_ref):
    o_sp = _dinvs(dp_sp_ref) * (ap_sp_ref[0] + ap_sp_ref[1] + z_sp_ref[...]) \
        + b2_sp_ref[...]
    o_si = _dinvs(dp_si_ref) * (ap_si_ref[0] + ap_si_ref[1] + z_si_ref[...]) \
        + b2_si_ref[...]
    out_ref[...] = (o_sp + o_si) * 0.5


def _rows(width):
    return pl.BlockSpec((BLK, width), lambda i: (i, 0))


def _part(width):
    return pl.BlockSpec((NC, BLK, width), lambda i: (0, i, 0))


def _full(r, c):
    return pl.BlockSpec((r, c), lambda i: (0, 0))


_GRID = (N // BLK,)

_tc_pre = pl.pallas_call(
    _tc_pre_body,
    grid=_GRID,
    in_specs=[_rows(IN_CH), _full(IN_CH, HID), _full(IN_CH, HID),
              _part(16), _part(16)],
    out_specs=[_rows(HID), _rows(HID)],
    out_shape=[jax.ShapeDtypeStruct((N, HID), jnp.float32)] * 2,
)

_tc_mid = pl.pallas_call(
    _tc_mid_body,
    grid=_GRID,
    in_specs=[_part(HID), _part(HID), _rows(HID), _rows(HID),
              _part(16), _part(16), _full(1, HID), _full(1, HID),
              _full(HID, OUT_PAD), _full(HID, OUT_PAD)],
    out_specs=[_rows(OUT_PAD), _rows(OUT_PAD)],
    out_shape=[jax.ShapeDtypeStruct((N, OUT_PAD), jnp.float32)] * 2,
)

_tc_final = pl.pallas_call(
    _tc_final_body,
    grid=_GRID,
    in_specs=[_part(OUT_PAD), _part(OUT_PAD), _rows(OUT_PAD), _rows(OUT_PAD),
              _part(16), _part(16), _full(1, OUT_PAD), _full(1, OUT_PAD)],
    out_specs=pl.BlockSpec((BLK, OUT_PAD), lambda i: (i, 0)),
    out_shape=jax.ShapeDtypeStruct((N, OUT_PAD), jnp.float32),
)


def kernel(x, spatial_edge_index, sim_edge_index,
           W1_spatial, b1_spatial, W1_sim, b1_sim,
           W2_spatial, b2_spatial, W2_sim, b2_sim):
    f32 = jnp.float32
    src_sp = spatial_edge_index[0].astype(jnp.int32).reshape(NW, NCHUNK, CH)
    dst_sp = spatial_edge_index[1].astype(jnp.int32).reshape(NW, NCHUNK, CH)
    src_si = sim_edge_index[0].astype(jnp.int32).reshape(NW, NCHUNK, CH)
    dst_si = sim_edge_index[1].astype(jnp.int32).reshape(NW, NCHUNK, CH)

    ones16 = jnp.ones((CH, 16), f32)
    zeros16 = jnp.zeros((RPT, 16), f32)
    zeros64 = jnp.zeros((RPT, HID), f32)

    w2p_sp = jnp.zeros((HID, OUT_PAD), f32).at[:, :10].set(W2_spatial)
    w2p_si = jnp.zeros((HID, OUT_PAD), f32).at[:, :10].set(W2_sim)
    b1r_sp = b1_spatial.reshape(1, HID)
    b1r_si = b1_sim.reshape(1, HID)
    b2p_sp = jnp.zeros((1, OUT_PAD), f32).at[0, :10].set(b2_spatial)
    b2p_si = jnp.zeros((1, OUT_PAD), f32).at[0, :10].set(b2_sim)

    degp_sp, degp_si = _sc_degree(dst_sp, dst_si, ones16, zeros16)
    y_sp, y_si = _tc_pre(x, W1_spatial, W1_sim, degp_sp, degp_si)
    aggp_sp, aggp_si = _sc_agg64(src_sp, dst_sp, src_si, dst_si,
                                 y_sp, y_si, zeros64)
    z_sp, z_si = _tc_mid(aggp_sp, aggp_si, y_sp, y_si, degp_sp, degp_si,
                         b1r_sp, b1r_si, w2p_sp, w2p_si)
    agg2p_sp, agg2p_si = _sc_agg16(src_sp, dst_sp, src_si, dst_si,
                                   z_sp, z_si, zeros16)
    outp = _tc_final(agg2p_sp, agg2p_si, z_sp, z_si, degp_sp, degp_si,
                     b2p_sp, b2p_si)
    return outp[:, :10]


# speedup vs baseline: 34.7148x; 34.7148x over previous
import functools

import jax
import jax.numpy as jnp
from jax import lax
from jax.experimental import pallas as pl
from jax.experimental.pallas import tpu as pltpu
from jax.experimental.pallas import tpu_sc as plsc

N = 10000
IN_CH = 128
HID = 64
OUT_PAD = 16
E = 320000

NC = 2
NS = 16
NW = NC * NS
EPT = E // NW
CH = 100
NCHUNK = EPT // CH
NP = 10240
RPT = NP // NS

BLK = 1000


def _mesh():
    return plsc.VectorSubcoreMesh(
        core_axis_name="c", subcore_axis_name="s",
        num_cores=NC, num_subcores=NS)


@functools.cache
def _make_sc_degree():
    @functools.partial(
        pl.kernel,
        out_type=(jax.ShapeDtypeStruct((NC, NP, 16), jnp.float32),
                  jax.ShapeDtypeStruct((NC, NP, 16), jnp.float32)),
        mesh=_mesh(),
        scratch_types=[
            pltpu.VMEM((NCHUNK, CH), jnp.int32),
            pltpu.VMEM((CH, 16), jnp.float32),
            pltpu.VMEM((RPT, 16), jnp.float32),
            pltpu.VMEM_SHARED((NP, 16), jnp.float32),
            pltpu.VMEM_SHARED((NP, 16), jnp.float32),
        ],
        compiler_params=pltpu.CompilerParams(use_tc_tiling_on_sc=False),
    )
    def _sc_degree(dst_sp, dst_si, ones_in, zeros_in, degp_sp, degp_si,
                   dst_v, ones_v, stage_v, acc_sp, acc_si):
        c = lax.axis_index("c")
        s = lax.axis_index("s")
        wid = c * NS + s
        row0 = s * RPT
        pltpu.sync_copy(zeros_in, stage_v)
        pltpu.sync_copy(stage_v, acc_sp.at[pl.ds(row0, RPT)])
        pltpu.sync_copy(stage_v, acc_si.at[pl.ds(row0, RPT)])
        pltpu.sync_copy(ones_in, ones_v)
        plsc.subcore_barrier()
        for dst_hbm, acc in ((dst_sp, acc_sp), (dst_si, acc_si)):
            pltpu.sync_copy(dst_hbm.at[wid], dst_v)

            @pl.loop(0, NCHUNK)
            def _(i, acc=acc):
                pltpu.sync_copy(ones_v, acc.at[dst_v.at[i]], add=True)

        plsc.subcore_barrier()
        for acc, out in ((acc_sp, degp_sp), (acc_si, degp_si)):
            pltpu.sync_copy(acc.at[pl.ds(row0, RPT)], stage_v)
            pltpu.sync_copy(stage_v, out.at[c, pl.ds(row0, RPT)])

    return _sc_degree


@functools.cache
def _make_sc_agg(width):
    @functools.partial(
        pl.kernel,
        out_type=(jax.ShapeDtypeStruct((NC, NP, width), jnp.float32),
                  jax.ShapeDtypeStruct((NC, NP, width), jnp.float32)),
        mesh=_mesh(),
        scratch_types=[
            pltpu.VMEM((NCHUNK, CH), jnp.int32),
            pltpu.VMEM((NCHUNK, CH), jnp.int32),
            pltpu.VMEM((CH, width), jnp.float32),
            pltpu.VMEM((CH, width), jnp.float32),
            pltpu.VMEM((RPT, width), jnp.float32),
            pltpu.VMEM_SHARED((NP, width), jnp.float32),
            pltpu.SemaphoreType.DMA,
            pltpu.SemaphoreType.DMA,
        ],
        compiler_params=pltpu.CompilerParams(use_tc_tiling_on_sc=False),
    )
    def _sc_agg(src_sp, dst_sp, src_si, dst_si, y_sp, y_si, zeros_in,
                aggp_sp, aggp_si,
                src_v, dst_v, rows0, rows1, stage_v, acc,
                sem0, sem1):
        c = lax.axis_index("c")
        s = lax.axis_index("s")
        wid = c * NS + s
        row0 = s * RPT
        pltpu.sync_copy(zeros_in, stage_v)
        for src_hbm, dst_hbm, y_hbm, out in (
                (src_sp, dst_sp, y_sp, aggp_sp),
                (src_si, dst_si, y_si, aggp_si)):
            pltpu.sync_copy(stage_v, acc.at[pl.ds(row0, RPT)])
            pltpu.sync_copy(src_hbm.at[wid], src_v)
            pltpu.sync_copy(dst_hbm.at[wid], dst_v)
            plsc.subcore_barrier()

            @pl.loop(0, NCHUNK, step=2)
            def _(i, y_hbm=y_hbm):
                cp0 = pltpu.async_copy(y_hbm.at[src_v.at[i]], rows0, sem0)
                cp1 = pltpu.async_copy(y_hbm.at[src_v.at[i + 1]], rows1, sem1)
                cp0.wait()
                pltpu.sync_copy(rows0, acc.at[dst_v.at[i]], add=True)
                cp1.wait()
                pltpu.sync_copy(rows1, acc.at[dst_v.at[i + 1]], add=True)

            plsc.subcore_barrier()
            pltpu.sync_copy(acc.at[pl.ds(row0, RPT)], stage_v)
            pltpu.sync_copy(stage_v, out.at[c, pl.ds(row0, RPT)])
            pltpu.sync_copy(zeros_in, stage_v)

    return _sc_agg


def _dinvs(dp_ref):
    deg = dp_ref[0, :, 0:1] + dp_ref[1, :, 0:1] + 1.0
    return lax.rsqrt(deg)


def _tc_pre_body(x_ref, w_sp_ref, w_si_ref, dp_sp_ref, dp_si_ref,
                 y_sp_ref, y_si_ref):
    xb = x_ref[...]
    y_sp_ref[...] = _dinvs(dp_sp_ref) * jnp.dot(
        xb, w_sp_ref[...], preferred_element_type=jnp.float32)
    y_si_ref[...] = _dinvs(dp_si_ref) * jnp.dot(
        xb, w_si_ref[...], preferred_element_type=jnp.float32)


def _tc_mid_body(ap_sp_ref, ap_si_ref, y_sp_ref, y_si_ref,
                 dp_sp_ref, dp_si_ref, b1_sp_ref, b1_si_ref,
                 w2_sp_ref, w2_si_ref, z_sp_ref, z_si_ref):
    di_sp = _dinvs(dp_sp_ref)
    di_si = _dinvs(dp_si_ref)
    h_sp = di_sp * (ap_sp_ref[0] + ap_sp_ref[1] + y_sp_ref[...]) + b1_sp_ref[...]
    h_si = di_si * (ap_si_ref[0] + ap_si_ref[1] + y_si_ref[...]) + b1_si_ref[...]
    h = jnp.maximum((h_sp + h_si) * 0.5, 0.0)
    z_sp_ref[...] = di_sp * jnp.dot(
        h, w2_sp_ref[...], preferred_element_type=jnp.float32)
    z_si_ref[...] = di_si * jnp.dot(
        h, w2_si_ref[...], preferred_element_type=jnp.float32)


def _tc_final_body(ap_sp_ref, ap_si_ref, z_sp_ref, z_si_ref,
                   dp_sp_ref, dp_si_ref, b2_sp_ref, b2_si_ref, out_ref):
    o_sp = _dinvs(dp_sp_ref) * (ap_sp_ref[0] + ap_sp_ref[1] + z_sp_ref[...]) \
        + b2_sp_ref[...]
    o_si = _dinvs(dp_si_ref) * (ap_si_ref[0] + ap_si_ref[1] + z_si_ref[...]) \
        + b2_si_ref[...]
    out_ref[...] = (o_sp + o_si) * 0.5


def _rows(width):
    return pl.BlockSpec((BLK, width), lambda i: (i, 0))


def _part(width):
    return pl.BlockSpec((NC, BLK, width), lambda i: (0, i, 0))


def _full(r, c):
    return pl.BlockSpec((r, c), lambda i: (0, 0))


_GRID = (N // BLK,)

_tc_pre = pl.pallas_call(
    _tc_pre_body,
    grid=_GRID,
    in_specs=[_rows(IN_CH), _full(IN_CH, HID), _full(IN_CH, HID),
              _part(16), _part(16)],
    out_specs=[_rows(HID), _rows(HID)],
    out_shape=[jax.ShapeDtypeStruct((N, HID), jnp.float32)] * 2,
)

_tc_mid = pl.pallas_call(
    _tc_mid_body,
    grid=_GRID,
    in_specs=[_part(HID), _part(HID), _rows(HID), _rows(HID),
              _part(16), _part(16), _full(1, HID), _full(1, HID),
              _full(HID, OUT_PAD), _full(HID, OUT_PAD)],
    out_specs=[_rows(OUT_PAD), _rows(OUT_PAD)],
    out_shape=[jax.ShapeDtypeStruct((N, OUT_PAD), jnp.float32)] * 2,
)

_tc_final = pl.pallas_call(
    _tc_final_body,
    grid=_GRID,
    in_specs=[_part(OUT_PAD), _part(OUT_PAD), _rows(OUT_PAD), _rows(OUT_PAD),
              _part(16), _part(16), _full(1, OUT_PAD), _full(1, OUT_PAD)],
    out_specs=pl.BlockSpec((BLK, OUT_PAD), lambda i: (i, 0)),
    out_shape=jax.ShapeDtypeStruct((N, OUT_PAD), jnp.float32),
)


def kernel(x, spatial_edge_index, sim_edge_index,
           W1_spatial, b1_spatial, W1_sim, b1_sim,
           W2_spatial, b2_spatial, W2_sim, b2_sim):
    f32 = jnp.float32
    src_sp = spatial_edge_index[0].astype(jnp.int32).reshape(NW, NCHUNK, CH)
    dst_sp = spatial_edge_index[1].astype(jnp.int32).reshape(NW, NCHUNK, CH)
    src_si = sim_edge_index[0].astype(jnp.int32).reshape(NW, NCHUNK, CH)
    dst_si = sim_edge_index[1].astype(jnp.int32).reshape(NW, NCHUNK, CH)

    ones16 = jnp.ones((CH, 16), f32)
    zeros16 = jnp.zeros((RPT, 16), f32)
    zeros64 = jnp.zeros((RPT, HID), f32)

    w2p_sp = jnp.zeros((HID, OUT_PAD), f32).at[:, :10].set(W2_spatial)
    w2p_si = jnp.zeros((HID, OUT_PAD), f32).at[:, :10].set(W2_sim)
    b1r_sp = b1_spatial.reshape(1, HID)
    b1r_si = b1_sim.reshape(1, HID)
    b2p_sp = jnp.zeros((1, OUT_PAD), f32).at[0, :10].set(b2_spatial)
    b2p_si = jnp.zeros((1, OUT_PAD), f32).at[0, :10].set(b2_sim)

    degp_sp, degp_si = _make_sc_degree()(dst_sp, dst_si, ones16, zeros16)
    y_sp, y_si = _tc_pre(x, W1_spatial, W1_sim, degp_sp, degp_si)
    aggp_sp, aggp_si = _make_sc_agg(HID)(src_sp, dst_sp, src_si, dst_si,
                                         y_sp, y_si, zeros64)
    z_sp, z_si = _tc_mid(aggp_sp, aggp_si, y_sp, y_si, degp_sp, degp_si,
                         b1r_sp, b1r_si, w2p_sp, w2p_si)
    agg2p_sp, agg2p_si = _make_sc_agg(OUT_PAD)(src_sp, dst_sp, src_si, dst_si,
                                               z_sp, z_si, zeros16)
    outp = _tc_final(agg2p_sp, agg2p_si, z_sp, z_si, degp_sp, degp_si,
                     b2p_sp, b2p_si)
    return outp[:, :10]


# speedup vs baseline: 49.3501x; 1.4216x over previous
import functools

import jax
import jax.numpy as jnp
from jax import lax
from jax.experimental import pallas as pl
from jax.experimental.pallas import tpu as pltpu
from jax.experimental.pallas import tpu_sc as plsc

N = 10000
IN_CH = 128
HID = 64
OUT_PAD = 16
E = 320000

NC = 2
NS = 16
NW = NC * NS
EPT = E // NW
CH = 125
NCHUNK = EPT // CH
NSLOT = 5
LAG = 2
NP = 10240
RPT = NP // NS

BLK = 1000


def _mesh():
    return plsc.VectorSubcoreMesh(
        core_axis_name="c", subcore_axis_name="s",
        num_cores=NC, num_subcores=NS)


@functools.cache
def _make_sc_degree():
    @functools.partial(
        pl.kernel,
        out_type=(jax.ShapeDtypeStruct((NC, NP, 16), jnp.float32),
                  jax.ShapeDtypeStruct((NC, NP, 16), jnp.float32)),
        mesh=_mesh(),
        scratch_types=[
            pltpu.VMEM((NCHUNK, CH), jnp.int32),
            pltpu.VMEM((CH, 16), jnp.float32),
            pltpu.VMEM_SHARED((NP, 16), jnp.float32),
            pltpu.VMEM_SHARED((NP, 16), jnp.float32),
            pltpu.SemaphoreType.DMA,
        ],
        compiler_params=pltpu.CompilerParams(use_tc_tiling_on_sc=False),
    )
    def _sc_degree(dst_sp, dst_si, ones_in, zeros_in, degp_sp, degp_si,
                   dst_v, ones_v, acc_sp, acc_si, dsem):
        c = lax.axis_index("c")
        s = lax.axis_index("s")
        wid = c * NS + s
        row0 = s * RPT
        pltpu.sync_copy(zeros_in, acc_sp.at[pl.ds(row0, RPT)])
        pltpu.sync_copy(zeros_in, acc_si.at[pl.ds(row0, RPT)])
        pltpu.sync_copy(ones_in, ones_v)
        plsc.subcore_barrier()
        dlag = 16
        for dst_hbm, acc in ((dst_sp, acc_sp), (dst_si, acc_si)):
            pltpu.sync_copy(dst_hbm.at[wid], dst_v)
            for i in range(dlag):
                pltpu.async_copy(ones_v, acc.at[dst_v.at[i]], dsem, add=True)

            @pl.loop(0, NCHUNK - dlag)
            def _(j, acc=acc):
                pltpu.make_async_copy(ones_v, acc.at[dst_v.at[j]], dsem).wait()
                pltpu.async_copy(
                    ones_v, acc.at[dst_v.at[j + dlag]], dsem, add=True)

            @pl.loop(NCHUNK - dlag, NCHUNK)
            def _(j, acc=acc):
                pltpu.make_async_copy(ones_v, acc.at[dst_v.at[j]], dsem).wait()

        plsc.subcore_barrier()
        pltpu.sync_copy(acc_sp.at[pl.ds(row0, RPT)],
                        degp_sp.at[c, pl.ds(row0, RPT)])
        pltpu.sync_copy(acc_si.at[pl.ds(row0, RPT)],
                        degp_si.at[c, pl.ds(row0, RPT)])

    return _sc_degree


@functools.cache
def _make_sc_agg(width):
    @functools.partial(
        pl.kernel,
        out_type=(jax.ShapeDtypeStruct((NC, NP, width), jnp.float32),
                  jax.ShapeDtypeStruct((NC, NP, width), jnp.float32)),
        mesh=_mesh(),
        scratch_types=[
            pltpu.VMEM((NCHUNK, CH), jnp.int32),
            pltpu.VMEM((NCHUNK, CH), jnp.int32),
            [pltpu.VMEM((CH, width), jnp.float32)] * NSLOT,
            pltpu.VMEM_SHARED((NP, width), jnp.float32),
            [pltpu.SemaphoreType.DMA] * NSLOT,
            [pltpu.SemaphoreType.DMA] * NSLOT,
        ],
        compiler_params=pltpu.CompilerParams(use_tc_tiling_on_sc=False),
    )
    def _sc_agg(src_sp, dst_sp, src_si, dst_si, y_sp, y_si, zeros_in,
                aggp_sp, aggp_si,
                src_v, dst_v, rows, acc, gsem, ssem):
        c = lax.axis_index("c")
        s = lax.axis_index("s")
        wid = c * NS + s
        row0 = s * RPT
        nrounds = NCHUNK // NSLOT
        for src_hbm, dst_hbm, y_hbm, out in (
                (src_sp, dst_sp, y_sp, aggp_sp),
                (src_si, dst_si, y_si, aggp_si)):
            pltpu.sync_copy(zeros_in, acc.at[pl.ds(row0, RPT)])
            pltpu.sync_copy(src_hbm.at[wid], src_v)
            pltpu.sync_copy(dst_hbm.at[wid], dst_v)
            plsc.subcore_barrier()

            def fire_g(i, r, y_hbm=y_hbm):
                pltpu.async_copy(y_hbm.at[src_v.at[i]], rows[r], gsem[r])

            def wait_g(i, r, y_hbm=y_hbm):
                pltpu.make_async_copy(
                    y_hbm.at[src_v.at[i]], rows[r], gsem[r]).wait()

            def fire_s(i, r):
                pltpu.async_copy(rows[r], acc.at[dst_v.at[i]], ssem[r],
                                 add=True)

            def wait_s(i, r):
                pltpu.make_async_copy(
                    rows[r], acc.at[dst_v.at[i]], ssem[r]).wait()

            for r in range(NSLOT):
                fire_g(r, r)
            for r in range(NSLOT):
                wait_g(r, r)
                fire_s(r, r)
                p = r - LAG
                if p >= 0:
                    wait_s(p, p % NSLOT)
                    fire_g(p + NSLOT, p % NSLOT)

            @pl.loop(1, nrounds - 1)
            def _(j, fire_g=fire_g, wait_g=wait_g):
                for r in range(NSLOT):
                    i = j * NSLOT + r
                    wait_g(i, r)
                    fire_s(i, r)
                    rp = (r - LAG) % NSLOT
                    wait_s(i - LAG, rp)
                    fire_g(i - LAG + NSLOT, rp)

            base = (nrounds - 1) * NSLOT
            for r in range(NSLOT):
                i = base + r
                wait_g(i, r)
                fire_s(i, r)
                p = i - LAG
                if p + NSLOT < NCHUNK:
                    wait_s(p, p % NSLOT)
                    fire_g(p + NSLOT, p % NSLOT)
            for i in range(NCHUNK - NSLOT, NCHUNK):
                wait_s(i, i % NSLOT)

            plsc.subcore_barrier()
            pltpu.sync_copy(acc.at[pl.ds(row0, RPT)],
                            out.at[c, pl.ds(row0, RPT)])

    return _sc_agg


def _dinvs(dp_ref):
    deg = dp_ref[0, :, 0:1] + dp_ref[1, :, 0:1] + 1.0
    return lax.rsqrt(deg)


def _tc_pre_body(x_ref, w_sp_ref, w_si_ref, dp_sp_ref, dp_si_ref,
                 y_sp_ref, y_si_ref):
    xb = x_ref[...]
    y_sp_ref[...] = _dinvs(dp_sp_ref) * jnp.dot(
        xb, w_sp_ref[...], preferred_element_type=jnp.float32)
    y_si_ref[...] = _dinvs(dp_si_ref) * jnp.dot(
        xb, w_si_ref[...], preferred_element_type=jnp.float32)


def _tc_mid_body(ap_sp_ref, ap_si_ref, y_sp_ref, y_si_ref,
                 dp_sp_ref, dp_si_ref, b1_sp_ref, b1_si_ref,
                 w2_sp_ref, w2_si_ref, z_sp_ref, z_si_ref):
    di_sp = _dinvs(dp_sp_ref)
    di_si = _dinvs(dp_si_ref)
    h_sp = di_sp * (ap_sp_ref[0] + ap_sp_ref[1] + y_sp_ref[...]) + b1_sp_ref[...]
    h_si = di_si * (ap_si_ref[0] + ap_si_ref[1] + y_si_ref[...]) + b1_si_ref[...]
    h = jnp.maximum((h_sp + h_si) * 0.5, 0.0)
    z_sp_ref[...] = di_sp * jnp.dot(
        h, w2_sp_ref[...], preferred_element_type=jnp.float32)
    z_si_ref[...] = di_si * jnp.dot(
        h, w2_si_ref[...], preferred_element_type=jnp.float32)


def _tc_final_body(ap_sp_ref, ap_si_ref, z_sp_ref, z_si_ref,
                   dp_sp_ref, dp_si_ref, b2_sp_ref, b2_si_ref, out_ref):
    o_sp = _dinvs(dp_sp_ref) * (ap_sp_ref[0] + ap_sp_ref[1] + z_sp_ref[...]) \
        + b2_sp_ref[...]
    o_si = _dinvs(dp_si_ref) * (ap_si_ref[0] + ap_si_ref[1] + z_si_ref[...]) \
        + b2_si_ref[...]
    out_ref[...] = (o_sp + o_si) * 0.5


def _rows(width):
    return pl.BlockSpec((BLK, width), lambda i: (i, 0))


def _part(width):
    return pl.BlockSpec((NC, BLK, width), lambda i: (0, i, 0))


def _full(r, c):
    return pl.BlockSpec((r, c), lambda i: (0, 0))


_GRID = (N // BLK,)

_tc_pre = pl.pallas_call(
    _tc_pre_body,
    grid=_GRID,
    in_specs=[_rows(IN_CH), _full(IN_CH, HID), _full(IN_CH, HID),
              _part(16), _part(16)],
    out_specs=[_rows(HID), _rows(HID)],
    out_shape=[jax.ShapeDtypeStruct((N, HID), jnp.float32)] * 2,
)

_tc_mid = pl.pallas_call(
    _tc_mid_body,
    grid=_GRID,
    in_specs=[_part(HID), _part(HID), _rows(HID), _rows(HID),
              _part(16), _part(16), _full(1, HID), _full(1, HID),
              _full(HID, OUT_PAD), _full(HID, OUT_PAD)],
    out_specs=[_rows(OUT_PAD), _rows(OUT_PAD)],
    out_shape=[jax.ShapeDtypeStruct((N, OUT_PAD), jnp.float32)] * 2,
)

_tc_final = pl.pallas_call(
    _tc_final_body,
    grid=_GRID,
    in_specs=[_part(OUT_PAD), _part(OUT_PAD), _rows(OUT_PAD), _rows(OUT_PAD),
              _part(16), _part(16), _full(1, OUT_PAD), _full(1, OUT_PAD)],
    out_specs=pl.BlockSpec((BLK, OUT_PAD), lambda i: (i, 0)),
    out_shape=jax.ShapeDtypeStruct((N, OUT_PAD), jnp.float32),
)


def kernel(x, spatial_edge_index, sim_edge_index,
           W1_spatial, b1_spatial, W1_sim, b1_sim,
           W2_spatial, b2_spatial, W2_sim, b2_sim):
    f32 = jnp.float32
    src_sp = spatial_edge_index[0].astype(jnp.int32).reshape(NW, NCHUNK, CH)
    dst_sp = spatial_edge_index[1].astype(jnp.int32).reshape(NW, NCHUNK, CH)
    src_si = sim_edge_index[0].astype(jnp.int32).reshape(NW, NCHUNK, CH)
    dst_si = sim_edge_index[1].astype(jnp.int32).reshape(NW, NCHUNK, CH)

    ones16 = jnp.ones((CH, 16), f32)
    zeros16 = jnp.zeros((RPT, 16), f32)
    zeros64 = jnp.zeros((RPT, HID), f32)

    w2p_sp = jnp.zeros((HID, OUT_PAD), f32).at[:, :10].set(W2_spatial)
    w2p_si = jnp.zeros((HID, OUT_PAD), f32).at[:, :10].set(W2_sim)
    b1r_sp = b1_spatial.reshape(1, HID)
    b1r_si = b1_sim.reshape(1, HID)
    b2p_sp = jnp.zeros((1, OUT_PAD), f32).at[0, :10].set(b2_spatial)
    b2p_si = jnp.zeros((1, OUT_PAD), f32).at[0, :10].set(b2_sim)

    degp_sp, degp_si = _make_sc_degree()(dst_sp, dst_si, ones16, zeros16)
    y_sp, y_si = _tc_pre(x, W1_spatial, W1_sim, degp_sp, degp_si)
    aggp_sp, aggp_si = _make_sc_agg(HID)(src_sp, dst_sp, src_si, dst_si,
                                         y_sp, y_si, zeros64)
    z_sp, z_si = _tc_mid(aggp_sp, aggp_si, y_sp, y_si, degp_sp, degp_si,
                         b1r_sp, b1r_si, w2p_sp, w2p_si)
    agg2p_sp, agg2p_si = _make_sc_agg(OUT_PAD)(src_sp, dst_sp, src_si, dst_si,
                                               z_sp, z_si, zeros16)
    outp = _tc_final(agg2p_sp, agg2p_si, z_sp, z_si, degp_sp, degp_si,
                     b2p_sp, b2p_si)
    return outp[:, :10]
